# Initial kernel scaffold; baseline (speedup 1.0000x reference)
import jax
import jax.numpy as jnp
from jax.experimental import pallas as pl


def kernel(x, conv_w, router_w, expert_w, expert_b, shared_w, shared_b):
    raise NotImplementedError("write your pallas kernel here")


# speedup vs baseline: 1.0749x; 1.0749x over previous
import functools

import jax
import jax.numpy as jnp
import numpy as np
from jax import lax
from jax.experimental import pallas as pl
from jax.experimental.pallas import tpu as pltpu
from jax.experimental.pallas import tpu_sc as plsc

IN_CH = 96
OUT_CH = 96
N_EXPERT = 64
TOP_K = 2
CAP_F = 1.25
MOE_DIM = 3 * IN_CH
RB = 8


def _stageA_body(nbpb, W, TB, xp_ref, xc_ref, xn_ref, wt_ref, rw_ref,
                 xf_ref, idx_ref, gate_ref, win_ref, hist_ref):
    g = pl.program_id(0)
    base_h = (g % nbpb) * RB
    xh = jnp.concatenate([xp_ref[...], xc_ref[...], xn_ref[...]], axis=0)
    xh = xh.astype(jnp.bfloat16).astype(jnp.float32)
    t_loc = lax.broadcasted_iota(jnp.int32, (TB, 1), 0)
    w_img = t_loc % W
    h_img = base_h + t_loc // W
    H_img = nbpb * RB
    mw = [(w_img > 0), None, (w_img < W - 1)]
    mh = [(h_img > 0), None, (h_img < H_img - 1)]
    acc = [jnp.zeros((TB, IN_CH), jnp.float32) for _ in range(3)]
    for dh in range(3):
        for dw in range(3):
            off = (dh - 1) * W + (dw - 1)
            sl = xh[TB + off:2 * TB + off, :]
            m = None
            if mh[dh] is not None:
                m = mh[dh]
            if mw[dw] is not None:
                m = mw[dw] if m is None else (m & mw[dw])
            if m is not None:
                sl = sl * m.astype(jnp.float32)
            for j in range(3):
                wv = wt_ref[j * 9 + dh * 3 + dw, :].astype(jnp.float32)
                acc[j] = acc[j] + sl * wv[None, :]
    xflat = jnp.concatenate(acc, axis=1)
    xf_ref[...] = xflat

    logits = jnp.dot(xflat, rw_ref[...], preferred_element_type=jnp.float32)
    mx = jnp.max(logits, axis=1, keepdims=True)
    ex = jnp.exp(logits - mx)
    gates = ex / jnp.sum(ex, axis=1, keepdims=True)

    lane = lax.broadcasted_iota(jnp.int32, (TB, N_EXPERT), 1)
    m1 = jnp.max(gates, axis=1, keepdims=True)
    i1 = jnp.min(jnp.where(gates == m1, lane, N_EXPERT), axis=1, keepdims=True)
    oh1 = (lane == i1)
    g2 = jnp.where(oh1, -1.0, gates)
    m2 = jnp.max(g2, axis=1, keepdims=True)
    i2 = jnp.min(jnp.where(g2 == m2, lane, N_EXPERT), axis=1, keepdims=True)
    oh2 = (lane == i2)
    den = m1 + m2 + 1e-9
    idx_ref[...] = jnp.concatenate([i1, i2], axis=1)
    gate_ref[...] = jnp.concatenate([m1 / den, m2 / den], axis=1)

    nch = TB // 128
    r_i = lax.broadcasted_iota(jnp.int32, (128, 128), 0)
    c_i = lax.broadcasted_iota(jnp.int32, (128, 128), 1)
    tri = (r_i > c_i).astype(jnp.bfloat16)

    def prefix(oh):
        ohf = oh.astype(jnp.float32)
        ohb = oh.astype(jnp.bfloat16)
        parts = []
        prev = jnp.zeros((1, N_EXPERT), jnp.float32)
        for q in range(nch):
            ohq = ohb[q * 128:(q + 1) * 128, :]
            exq = jnp.dot(tri, ohq, preferred_element_type=jnp.float32) + prev
            prev = prev + jnp.sum(ohf[q * 128:(q + 1) * 128, :], axis=0,
                                  keepdims=True)
            parts.append(exq)
        exf = jnp.concatenate(parts, axis=0)
        win = jnp.sum(exf * ohf, axis=1, keepdims=True)
        return win, prev

    win1, hist1 = prefix(oh1)
    win2, hist2 = prefix(oh2)
    win_ref[...] = jnp.concatenate([win1, win2], axis=1).astype(jnp.int32)
    hist_ref[...] = jnp.concatenate([hist1, hist2], axis=0)[None]


def _stageA(xT, wt, rw, T, W, nbpb, nblk, TB):
    grid = (nblk,)
    specs = [
        pl.BlockSpec((TB, IN_CH), lambda g: (jnp.maximum(g - 1, 0), 0)),
        pl.BlockSpec((TB, IN_CH), lambda g: (g, 0)),
        pl.BlockSpec((TB, IN_CH), lambda g: (jnp.minimum(g + 1, nblk - 1), 0)),
        pl.BlockSpec((27, IN_CH), lambda g: (0, 0)),
        pl.BlockSpec((MOE_DIM, N_EXPERT), lambda g: (0, 0)),
    ]
    outs = [
        jax.ShapeDtypeStruct((T, MOE_DIM), jnp.float32),
        jax.ShapeDtypeStruct((T, 2), jnp.int32),
        jax.ShapeDtypeStruct((T, 2), jnp.float32),
        jax.ShapeDtypeStruct((T, 2), jnp.int32),
        jax.ShapeDtypeStruct((nblk, 2, N_EXPERT), jnp.float32),
    ]
    out_specs = [
        pl.BlockSpec((TB, MOE_DIM), lambda g: (g, 0)),
        pl.BlockSpec((TB, 2), lambda g: (g, 0)),
        pl.BlockSpec((TB, 2), lambda g: (g, 0)),
        pl.BlockSpec((TB, 2), lambda g: (g, 0)),
        pl.BlockSpec((1, 2, N_EXPERT), lambda g: (g, 0, 0)),
    ]
    return pl.pallas_call(
        functools.partial(_stageA_body, nbpb, W, TB),
        grid=grid, in_specs=specs, out_specs=out_specs, out_shape=outs,
    )(xT, xT, xT, wt, rw)


def _stageB_body(C, TB, nblk, hist_ref, idx_ref, win_ref, gate_ref,
                 slot_ref, geff_ref):
    g = pl.program_id(0)
    hist0 = hist_ref[:, 0, :]
    hist1 = hist_ref[:, 1, :]
    blk = lax.broadcasted_iota(jnp.int32, (nblk, 1), 0)
    mprev = (blk < g).astype(jnp.float32)
    off0 = jnp.sum(hist0 * mprev, axis=0, keepdims=True)
    total0 = jnp.sum(hist0, axis=0, keepdims=True)
    off1 = total0 + jnp.sum(hist1 * mprev, axis=0, keepdims=True)

    idx = idx_ref[...]
    lane = lax.broadcasted_iota(jnp.int32, (TB, N_EXPERT), 1)
    oh1 = (lane == idx[:, 0:1]).astype(jnp.float32)
    oh2 = (lane == idx[:, 1:2]).astype(jnp.float32)
    base1 = jnp.sum(oh1 * off0, axis=1, keepdims=True)
    base2 = jnp.sum(oh2 * off1, axis=1, keepdims=True)
    win = win_ref[...]
    loc1 = win[:, 0:1].astype(jnp.float32) + base1
    loc2 = win[:, 1:2].astype(jnp.float32) + base2
    keep1 = (loc1 < C).astype(jnp.float32)
    keep2 = (loc2 < C).astype(jnp.float32)
    slot1 = jnp.minimum(loc1, C).astype(jnp.int32)
    slot2 = jnp.minimum(loc2, C).astype(jnp.int32)
    gate = gate_ref[...]
    slot_ref[...] = jnp.concatenate([slot1, slot2], axis=1)
    geff_ref[...] = jnp.concatenate(
        [gate[:, 0:1] * keep1, gate[:, 1:2] * keep2], axis=1)


def _stageB(hist, idx, win, gate, C, T, TB, nblk):
    specs = [
        pl.BlockSpec((nblk, 2, N_EXPERT), lambda g: (0, 0, 0)),
        pl.BlockSpec((TB, 2), lambda g: (g, 0)),
        pl.BlockSpec((TB, 2), lambda g: (g, 0)),
        pl.BlockSpec((TB, 2), lambda g: (g, 0)),
    ]
    outs = [
        jax.ShapeDtypeStruct((T, 2), jnp.int32),
        jax.ShapeDtypeStruct((T, 2), jnp.float32),
    ]
    out_specs = [
        pl.BlockSpec((TB, 2), lambda g: (g, 0)),
        pl.BlockSpec((TB, 2), lambda g: (g, 0)),
    ]
    return pl.pallas_call(
        functools.partial(_stageB_body, C, TB, nblk),
        grid=(nblk,), in_specs=specs, out_specs=out_specs, out_shape=outs,
    )(hist, idx, win, gate)


def _stageC(e_all, s_all, t_all, C, Cp, Np, n_assign):
    info = plsc.get_sparse_core_info()
    NW = info.num_cores * info.num_subcores
    per_w = n_assign // NW
    nv = per_w // 16
    mesh = plsc.VectorSubcoreMesh(core_axis_name="c", subcore_axis_name="s")

    @functools.partial(
        pl.kernel, mesh=mesh,
        compiler_params=pltpu.CompilerParams(use_tc_tiling_on_sc=False),
        out_type=jax.ShapeDtypeStruct((Np + 16,), jnp.int32),
        scratch_types=[
            pltpu.VMEM((per_w,), jnp.int32),
            pltpu.VMEM((per_w,), jnp.int32),
            pltpu.VMEM((per_w,), jnp.int32),
            pltpu.VMEM((128,), jnp.int32),
        ],
    )
    def k(e_hbm, s_hbm, t_hbm, tfs_hbm, e_v, s_v, t_v, addr_v):
        wid = lax.axis_index("s") * info.num_cores + lax.axis_index("c")
        base = wid * per_w
        pltpu.sync_copy(e_hbm.at[pl.ds(base, per_w)], e_v)
        pltpu.sync_copy(s_hbm.at[pl.ds(base, per_w)], s_v)
        pltpu.sync_copy(t_hbm.at[pl.ds(base, per_w)], t_v)

        def chunk(j, _):
            def body(i, _):
                sl = pl.ds(j * 128 + i * 16, 16)
                e = e_v[sl]
                s = s_v[sl]
                keep = s < C
                addr_v[pl.ds(i * 16, 16)] = jnp.where(keep, e * Cp + s, Np)
                return 0

            lax.fori_loop(0, 8, body, 0, unroll=True)
            pltpu.sync_copy(t_v.at[pl.ds(j * 128, 128)], tfs_hbm.at[addr_v])
            return 0

        lax.fori_loop(0, per_w // 128, chunk, 0)

    return k(e_all, s_all, t_all)


def _stageD(tfs, xf, T, Np):
    info = plsc.get_sparse_core_info()
    NW = info.num_cores * info.num_subcores
    per_w = Np // NW
    CH = 128
    nch = per_w // CH
    mesh = plsc.VectorSubcoreMesh(core_axis_name="c", subcore_axis_name="s")

    @functools.partial(
        pl.kernel, mesh=mesh,
        compiler_params=pltpu.CompilerParams(use_tc_tiling_on_sc=False),
        out_type=jax.ShapeDtypeStruct((Np, MOE_DIM), jnp.float32),
        scratch_types=[
            pltpu.VMEM((CH,), jnp.int32),
            pltpu.VMEM((CH,), jnp.int32),
            pltpu.VMEM((CH, MOE_DIM), jnp.float32),
            pltpu.SemaphoreType.DMA,
        ],
    )
    def k(tfs_hbm, xf_hbm, disp_hbm, raw_v, idx_v, rows_v, sem):
        wid = lax.axis_index("s") * info.num_cores + lax.axis_index("c")
        wbase = wid * per_w

        def body(ci, _):
            base = wbase + ci * CH
            pltpu.sync_copy(tfs_hbm.at[pl.ds(base, CH)], raw_v)

            def clampv(i, _):
                sl = pl.ds(i * 16, 16)
                idx_v[sl] = jnp.clip(raw_v[sl], 0, T - 1)
                return 0

            lax.fori_loop(0, CH // 16, clampv, 0)
            pltpu.async_copy(xf_hbm.at[idx_v], rows_v, sem).wait()
            pltpu.sync_copy(rows_v, disp_hbm.at[pl.ds(base, CH)])
            return 0

        lax.fori_loop(0, nch, body, 0)

    return k(tfs, xf)


def _stageE_body(disp_ref, ew_ref, eb_ref, h_ref):
    h = jnp.dot(disp_ref[...], ew_ref[0], preferred_element_type=jnp.float32)
    h_ref[...] = h + eb_ref[0]


def _stageE(disp, ew, eb, Cp, Np):
    CB = 496 if Cp % 496 == 0 else 128
    nb = Cp // CB
    grid = (N_EXPERT, nb)
    specs = [
        pl.BlockSpec((CB, MOE_DIM), lambda e, c: (e * nb + c, 0)),
        pl.BlockSpec((1, MOE_DIM, OUT_CH), lambda e, c: (e, 0, 0)),
        pl.BlockSpec((1, 1, OUT_CH), lambda e, c: (e, 0, 0)),
    ]
    out_spec = pl.BlockSpec((CB, OUT_CH), lambda e, c: (e * nb + c, 0))
    return pl.pallas_call(
        _stageE_body, grid=grid, in_specs=specs, out_specs=out_spec,
        out_shape=jax.ShapeDtypeStruct((Np, OUT_CH), jnp.float32),
    )(disp, ew, eb)


def _stageG(e_all, s_all, h, C, Cp, Np, n_assign):
    info = plsc.get_sparse_core_info()
    NW = info.num_cores * info.num_subcores
    per_w = n_assign // NW
    CH = 128
    nch = per_w // CH
    mesh = plsc.VectorSubcoreMesh(core_axis_name="c", subcore_axis_name="s")

    @functools.partial(
        pl.kernel, mesh=mesh,
        compiler_params=pltpu.CompilerParams(use_tc_tiling_on_sc=False),
        out_type=jax.ShapeDtypeStruct((n_assign, OUT_CH), jnp.float32),
        scratch_types=[
            pltpu.VMEM((CH,), jnp.int32),
            pltpu.VMEM((CH,), jnp.int32),
            pltpu.VMEM((CH,), jnp.int32),
            pltpu.VMEM((CH, OUT_CH), jnp.float32),
            pltpu.SemaphoreType.DMA,
        ],
    )
    def k(e_hbm, s_hbm, h_hbm, hg_hbm, e_v, s_v, r_v, rows_v, sem):
        wid = lax.axis_index("s") * info.num_cores + lax.axis_index("c")
        wbase = wid * per_w

        def body(ci, _):
            base = wbase + ci * CH
            pltpu.sync_copy(e_hbm.at[pl.ds(base, CH)], e_v)
            pltpu.sync_copy(s_hbm.at[pl.ds(base, CH)], s_v)

            def addr(i, _):
                sl = pl.ds(i * 16, 16)
                r_v[sl] = e_v[sl] * Cp + jnp.minimum(s_v[sl], C - 1)
                return 0

            lax.fori_loop(0, CH // 16, addr, 0)
            pltpu.async_copy(h_hbm.at[r_v], rows_v, sem).wait()
            pltpu.sync_copy(rows_v, hg_hbm.at[pl.ds(base, CH)])
            return 0

        lax.fori_loop(0, nch, body, 0)

    return k(e_all, s_all, h)


def _stageF_body(xf_ref, geff_ref, hg1_ref, hg2_ref, sw_ref, sb_ref, y_ref):
    y = jnp.dot(xf_ref[...], sw_ref[...], preferred_element_type=jnp.float32)
    ge = geff_ref[...]
    y = y + sb_ref[...] + ge[:, 0:1] * hg1_ref[...] + ge[:, 1:2] * hg2_ref[...]
    y_ref[...] = y


def _stageF(xf, geff, hg_all, sw, sb, T, TB, nblk):
    specs = [
        pl.BlockSpec((TB, MOE_DIM), lambda g: (g, 0)),
        pl.BlockSpec((TB, 2), lambda g: (g, 0)),
        pl.BlockSpec((TB, OUT_CH), lambda g: (g, 0)),
        pl.BlockSpec((TB, OUT_CH), lambda g: (nblk + g, 0)),
        pl.BlockSpec((MOE_DIM, OUT_CH), lambda g: (0, 0)),
        pl.BlockSpec((1, OUT_CH), lambda g: (0, 0)),
    ]
    out_spec = pl.BlockSpec((TB, OUT_CH), lambda g: (g, 0))
    return pl.pallas_call(
        _stageF_body, grid=(nblk,), in_specs=specs, out_specs=out_spec,
        out_shape=jax.ShapeDtypeStruct((T, OUT_CH), jnp.float32),
    )(xf, geff, hg_all, hg_all, sw, sb)


def kernel(x, conv_w, router_w, expert_w, expert_b, shared_w, shared_b):
    B, Cin, H, W = x.shape
    T = B * H * W
    C = int(CAP_F * TOP_K * T / N_EXPERT)
    Cp = ((C + 127) // 128) * 128
    Np = N_EXPERT * Cp
    TB = RB * W
    nbpb = H // RB
    nblk = B * nbpb
    n_assign = TOP_K * T

    cc = np.arange(MOE_DIM) % IN_CH
    jj = np.arange(MOE_DIM) // IN_CH
    perm = 3 * cc + jj
    rw = router_w[perm]
    ew = expert_w[:, perm, :]
    sw = shared_w[perm]
    wt = jnp.transpose(conv_w[:, 0].reshape(IN_CH, 3, 3, 3), (1, 2, 3, 0))
    wt = wt.reshape(27, IN_CH).astype(jnp.bfloat16)

    xT = jnp.transpose(x, (0, 2, 3, 1)).reshape(T, Cin)
    xf, idx, gate, win, hist = _stageA(xT, wt, rw, T, W, nbpb, nblk, TB)
    slots, geff = _stageB(hist, idx, win, gate, C, T, TB, nblk)

    e_all = jnp.transpose(idx).reshape(n_assign)
    s_all = jnp.transpose(slots).reshape(n_assign)
    t_all = jnp.concatenate([jnp.arange(T, dtype=jnp.int32)] * TOP_K)

    tfs = _stageC(e_all, s_all, t_all, C, Cp, Np, n_assign)
    disp = _stageD(tfs[:Np], xf, T, Np)
    h = _stageE(disp, ew, expert_b.reshape(N_EXPERT, 1, OUT_CH), Cp, Np)
    hg_all = _stageG(e_all, s_all, h, C, Cp, Np, n_assign)
    y_flat = _stageF(xf, geff, hg_all, sw, shared_b.reshape(1, OUT_CH),
                     T, TB, nblk)

    return jnp.transpose(y_flat.reshape(B, H, W, OUT_CH), (0, 3, 1, 2))


# speedup vs baseline: 1.1666x; 1.0853x over previous
import functools

import jax
import jax.numpy as jnp
import numpy as np
from jax import lax
from jax.experimental import pallas as pl
from jax.experimental.pallas import tpu as pltpu
from jax.experimental.pallas import tpu_sc as plsc

IN_CH = 96
OUT_CH = 96
N_EXPERT = 64
TOP_K = 2
CAP_F = 1.25
MOE_DIM = 3 * IN_CH
RB = 8


def _stageA_body(nbpb, W, TB, xp_ref, xc_ref, xn_ref, wt_ref, rw_ref,
                 xf_ref, idx_ref, gate_ref, win_ref, hist_ref):
    g = pl.program_id(0)
    base_h = (g % nbpb) * RB
    xh = jnp.concatenate([xp_ref[...], xc_ref[...], xn_ref[...]], axis=0)
    xh = xh.astype(jnp.bfloat16).astype(jnp.float32)
    t_loc = lax.broadcasted_iota(jnp.int32, (TB, 1), 0)
    w_img = t_loc % W
    h_img = base_h + t_loc // W
    H_img = nbpb * RB
    mw = [(w_img > 0), None, (w_img < W - 1)]
    mh = [(h_img > 0), None, (h_img < H_img - 1)]
    acc = [jnp.zeros((TB, IN_CH), jnp.float32) for _ in range(3)]
    for dh in range(3):
        for dw in range(3):
            off = (dh - 1) * W + (dw - 1)
            sl = xh[TB + off:2 * TB + off, :]
            m = None
            if mh[dh] is not None:
                m = mh[dh]
            if mw[dw] is not None:
                m = mw[dw] if m is None else (m & mw[dw])
            if m is not None:
                sl = sl * m.astype(jnp.float32)
            for j in range(3):
                wv = wt_ref[j * 9 + dh * 3 + dw, :].astype(jnp.float32)
                acc[j] = acc[j] + sl * wv[None, :]
    xflat = jnp.concatenate(acc, axis=1)
    xf_ref[...] = xflat

    logits = jnp.dot(xflat, rw_ref[...], preferred_element_type=jnp.float32)
    mx = jnp.max(logits, axis=1, keepdims=True)
    ex = jnp.exp(logits - mx)
    gates = ex / jnp.sum(ex, axis=1, keepdims=True)

    lane = lax.broadcasted_iota(jnp.int32, (TB, N_EXPERT), 1)
    m1 = jnp.max(gates, axis=1, keepdims=True)
    i1 = jnp.min(jnp.where(gates == m1, lane, N_EXPERT), axis=1, keepdims=True)
    oh1 = (lane == i1)
    g2 = jnp.where(oh1, -1.0, gates)
    m2 = jnp.max(g2, axis=1, keepdims=True)
    i2 = jnp.min(jnp.where(g2 == m2, lane, N_EXPERT), axis=1, keepdims=True)
    oh2 = (lane == i2)
    den = m1 + m2 + 1e-9
    idx_ref[...] = jnp.concatenate([i1, i2], axis=1)
    gate_ref[...] = jnp.concatenate([m1 / den, m2 / den], axis=1)

    nch = TB // 128
    r_i = lax.broadcasted_iota(jnp.int32, (128, 128), 0)
    c_i = lax.broadcasted_iota(jnp.int32, (128, 128), 1)
    tri = (r_i > c_i).astype(jnp.bfloat16)

    def prefix(oh):
        ohf = oh.astype(jnp.float32)
        ohb = oh.astype(jnp.bfloat16)
        parts = []
        prev = jnp.zeros((1, N_EXPERT), jnp.float32)
        for q in range(nch):
            ohq = ohb[q * 128:(q + 1) * 128, :]
            exq = jnp.dot(tri, ohq, preferred_element_type=jnp.float32) + prev
            prev = prev + jnp.sum(ohf[q * 128:(q + 1) * 128, :], axis=0,
                                  keepdims=True)
            parts.append(exq)
        exf = jnp.concatenate(parts, axis=0)
        win = jnp.sum(exf * ohf, axis=1, keepdims=True)
        return win, prev

    win1, hist1 = prefix(oh1)
    win2, hist2 = prefix(oh2)
    win_ref[...] = jnp.concatenate([win1, win2], axis=1).astype(jnp.int32)
    hist_ref[...] = jnp.concatenate([hist1, hist2], axis=0)[None]


def _stageA(xT, wt, rw, T, W, nbpb, nblk, TB):
    grid = (nblk,)
    specs = [
        pl.BlockSpec((TB, IN_CH), lambda g: (jnp.maximum(g - 1, 0), 0)),
        pl.BlockSpec((TB, IN_CH), lambda g: (g, 0)),
        pl.BlockSpec((TB, IN_CH), lambda g: (jnp.minimum(g + 1, nblk - 1), 0)),
        pl.BlockSpec((27, IN_CH), lambda g: (0, 0)),
        pl.BlockSpec((MOE_DIM, N_EXPERT), lambda g: (0, 0)),
    ]
    outs = [
        jax.ShapeDtypeStruct((T, MOE_DIM), jnp.float32),
        jax.ShapeDtypeStruct((T, 2), jnp.int32),
        jax.ShapeDtypeStruct((T, 2), jnp.float32),
        jax.ShapeDtypeStruct((T, 2), jnp.int32),
        jax.ShapeDtypeStruct((nblk, 2, N_EXPERT), jnp.float32),
    ]
    out_specs = [
        pl.BlockSpec((TB, MOE_DIM), lambda g: (g, 0)),
        pl.BlockSpec((TB, 2), lambda g: (g, 0)),
        pl.BlockSpec((TB, 2), lambda g: (g, 0)),
        pl.BlockSpec((TB, 2), lambda g: (g, 0)),
        pl.BlockSpec((1, 2, N_EXPERT), lambda g: (g, 0, 0)),
    ]
    return pl.pallas_call(
        functools.partial(_stageA_body, nbpb, W, TB),
        grid=grid, in_specs=specs, out_specs=out_specs, out_shape=outs,
    )(xT, xT, xT, wt, rw)


def _stageB_body(C, TB, nblk, hist_ref, idx_ref, win_ref, gate_ref,
                 slot_ref, geff_ref):
    g = pl.program_id(0)
    hist0 = hist_ref[:, 0, :]
    hist1 = hist_ref[:, 1, :]
    blk = lax.broadcasted_iota(jnp.int32, (nblk, 1), 0)
    mprev = (blk < g).astype(jnp.float32)
    off0 = jnp.sum(hist0 * mprev, axis=0, keepdims=True)
    total0 = jnp.sum(hist0, axis=0, keepdims=True)
    off1 = total0 + jnp.sum(hist1 * mprev, axis=0, keepdims=True)

    idx = idx_ref[...]
    lane = lax.broadcasted_iota(jnp.int32, (TB, N_EXPERT), 1)
    oh1 = (lane == idx[:, 0:1]).astype(jnp.float32)
    oh2 = (lane == idx[:, 1:2]).astype(jnp.float32)
    base1 = jnp.sum(oh1 * off0, axis=1, keepdims=True)
    base2 = jnp.sum(oh2 * off1, axis=1, keepdims=True)
    win = win_ref[...]
    loc1 = win[:, 0:1].astype(jnp.float32) + base1
    loc2 = win[:, 1:2].astype(jnp.float32) + base2
    keep1 = (loc1 < C).astype(jnp.float32)
    keep2 = (loc2 < C).astype(jnp.float32)
    slot1 = jnp.minimum(loc1, C).astype(jnp.int32)
    slot2 = jnp.minimum(loc2, C).astype(jnp.int32)
    gate = gate_ref[...]
    slot_ref[...] = jnp.concatenate([slot1, slot2], axis=1)
    geff_ref[...] = jnp.concatenate(
        [gate[:, 0:1] * keep1, gate[:, 1:2] * keep2], axis=1)


def _stageB(hist, idx, win, gate, C, T, TB, nblk):
    specs = [
        pl.BlockSpec((nblk, 2, N_EXPERT), lambda g: (0, 0, 0)),
        pl.BlockSpec((TB, 2), lambda g: (g, 0)),
        pl.BlockSpec((TB, 2), lambda g: (g, 0)),
        pl.BlockSpec((TB, 2), lambda g: (g, 0)),
    ]
    outs = [
        jax.ShapeDtypeStruct((T, 2), jnp.int32),
        jax.ShapeDtypeStruct((T, 2), jnp.float32),
    ]
    out_specs = [
        pl.BlockSpec((TB, 2), lambda g: (g, 0)),
        pl.BlockSpec((TB, 2), lambda g: (g, 0)),
    ]
    return pl.pallas_call(
        functools.partial(_stageB_body, C, TB, nblk),
        grid=(nblk,), in_specs=specs, out_specs=out_specs, out_shape=outs,
    )(hist, idx, win, gate)


def _stageC(e_all, s_all, t_all, C, Cp, Np, n_assign):
    info = plsc.get_sparse_core_info()
    NW = info.num_cores * info.num_subcores
    per_w = n_assign // NW
    nv = per_w // 16
    mesh = plsc.VectorSubcoreMesh(core_axis_name="c", subcore_axis_name="s")

    @functools.partial(
        pl.kernel, mesh=mesh,
        compiler_params=pltpu.CompilerParams(use_tc_tiling_on_sc=False),
        out_type=jax.ShapeDtypeStruct((Np + 16,), jnp.int32),
        scratch_types=[
            pltpu.VMEM((per_w,), jnp.int32),
            pltpu.VMEM((per_w,), jnp.int32),
            pltpu.VMEM((per_w,), jnp.int32),
            pltpu.VMEM((128,), jnp.int32),
        ],
    )
    def k(e_hbm, s_hbm, t_hbm, tfs_hbm, e_v, s_v, t_v, addr_v):
        wid = lax.axis_index("s") * info.num_cores + lax.axis_index("c")
        base = wid * per_w
        pltpu.sync_copy(e_hbm.at[pl.ds(base, per_w)], e_v)
        pltpu.sync_copy(s_hbm.at[pl.ds(base, per_w)], s_v)
        pltpu.sync_copy(t_hbm.at[pl.ds(base, per_w)], t_v)

        def chunk(j, _):
            def body(i, _):
                sl = pl.ds(j * 128 + i * 16, 16)
                e = e_v[sl]
                s = s_v[sl]
                keep = s < C
                addr_v[pl.ds(i * 16, 16)] = jnp.where(keep, e * Cp + s, Np)
                return 0

            lax.fori_loop(0, 8, body, 0, unroll=True)
            pltpu.sync_copy(t_v.at[pl.ds(j * 128, 128)], tfs_hbm.at[addr_v])
            return 0

        lax.fori_loop(0, per_w // 128, chunk, 0)

    return k(e_all, s_all, t_all)


def _stageD(tfs, xf, T, Np):
    info = plsc.get_sparse_core_info()
    NW = info.num_cores * info.num_subcores
    per_w = Np // NW
    CH = 128
    nch = per_w // CH
    mesh = plsc.VectorSubcoreMesh(core_axis_name="c", subcore_axis_name="s")

    @functools.partial(
        pl.kernel, mesh=mesh,
        compiler_params=pltpu.CompilerParams(use_tc_tiling_on_sc=False),
        out_type=jax.ShapeDtypeStruct((Np, MOE_DIM), jnp.float32),
        scratch_types=[
            pltpu.VMEM((per_w,), jnp.int32),
            pltpu.VMEM((CH,), jnp.int32),
            pltpu.VMEM((CH,), jnp.int32),
            pltpu.VMEM((CH, MOE_DIM), jnp.float32),
            pltpu.VMEM((CH, MOE_DIM), jnp.float32),
            pltpu.SemaphoreType.DMA,
            pltpu.SemaphoreType.DMA,
        ],
    )
    def k(tfs_hbm, xf_hbm, disp_hbm, raw_v, idx0, idx1, rows0, rows1,
          semA, semB):
        wid = lax.axis_index("s") * info.num_cores + lax.axis_index("c")
        wbase = wid * per_w
        pltpu.sync_copy(tfs_hbm.at[pl.ds(wbase, per_w)], raw_v)

        def clampv(i, _):
            sl = pl.ds(i * 16, 16)
            raw_v[sl] = jnp.clip(raw_v[sl], 0, T - 1)
            return 0

        lax.fori_loop(0, per_w // 16, clampv, 0)

        def body(p, _):
            b0 = 2 * p * CH
            b1 = (2 * p + 1) * CH

            def cp(i, _):
                sl = pl.ds(i * 16, 16)
                idx0[sl] = raw_v[pl.ds(b0 + i * 16, 16)]
                idx1[sl] = raw_v[pl.ds(b1 + i * 16, 16)]
                return 0

            lax.fori_loop(0, CH // 16, cp, 0)
            g0 = pltpu.async_copy(xf_hbm.at[idx0], rows0, semA)
            g1 = pltpu.async_copy(xf_hbm.at[idx1], rows1, semB)
            g0.wait()
            w0 = pltpu.async_copy(rows0, disp_hbm.at[pl.ds(wbase + b0, CH)],
                                  semA)
            g1.wait()
            w1 = pltpu.async_copy(rows1, disp_hbm.at[pl.ds(wbase + b1, CH)],
                                  semB)
            w0.wait()
            w1.wait()
            return 0

        lax.fori_loop(0, nch // 2, body, 0)

    return k(tfs, xf)


def _stageE_body(disp_ref, ew_ref, eb_ref, h_ref):
    h = jnp.dot(disp_ref[...], ew_ref[0], preferred_element_type=jnp.float32)
    h_ref[...] = h + eb_ref[0]


def _stageE(disp, ew, eb, Cp, Np):
    CB = 496 if Cp % 496 == 0 else 128
    nb = Cp // CB
    grid = (N_EXPERT, nb)
    specs = [
        pl.BlockSpec((CB, MOE_DIM), lambda e, c: (e * nb + c, 0)),
        pl.BlockSpec((1, MOE_DIM, OUT_CH), lambda e, c: (e, 0, 0)),
        pl.BlockSpec((1, 1, OUT_CH), lambda e, c: (e, 0, 0)),
    ]
    out_spec = pl.BlockSpec((CB, OUT_CH), lambda e, c: (e * nb + c, 0))
    return pl.pallas_call(
        _stageE_body, grid=grid, in_specs=specs, out_specs=out_spec,
        out_shape=jax.ShapeDtypeStruct((Np, OUT_CH), jnp.float32),
    )(disp, ew, eb)


def _stageG(e_all, s_all, h, C, Cp, Np, n_assign):
    info = plsc.get_sparse_core_info()
    NW = info.num_cores * info.num_subcores
    per_w = n_assign // NW
    CH = 128
    nch = per_w // CH
    mesh = plsc.VectorSubcoreMesh(core_axis_name="c", subcore_axis_name="s")

    @functools.partial(
        pl.kernel, mesh=mesh,
        compiler_params=pltpu.CompilerParams(use_tc_tiling_on_sc=False),
        out_type=jax.ShapeDtypeStruct((n_assign, OUT_CH), jnp.float32),
        scratch_types=[
            pltpu.VMEM((CH,), jnp.int32),
            pltpu.VMEM((CH,), jnp.int32),
            pltpu.VMEM((CH,), jnp.int32),
            pltpu.VMEM((CH, OUT_CH), jnp.float32),
            pltpu.SemaphoreType.DMA,
        ],
    )
    def k(e_hbm, s_hbm, h_hbm, hg_hbm, e_v, s_v, r_v, rows_v, sem):
        wid = lax.axis_index("s") * info.num_cores + lax.axis_index("c")
        wbase = wid * per_w

        def body(ci, _):
            base = wbase + ci * CH
            pltpu.sync_copy(e_hbm.at[pl.ds(base, CH)], e_v)
            pltpu.sync_copy(s_hbm.at[pl.ds(base, CH)], s_v)

            def addr(i, _):
                sl = pl.ds(i * 16, 16)
                r_v[sl] = e_v[sl] * Cp + jnp.minimum(s_v[sl], C - 1)
                return 0

            lax.fori_loop(0, CH // 16, addr, 0)
            pltpu.async_copy(h_hbm.at[r_v], rows_v, sem).wait()
            pltpu.sync_copy(rows_v, hg_hbm.at[pl.ds(base, CH)])
            return 0

        lax.fori_loop(0, nch, body, 0)

    return k(e_all, s_all, h)


def _stageF_body(xf_ref, geff_ref, hg1_ref, hg2_ref, sw_ref, sb_ref, y_ref):
    y = jnp.dot(xf_ref[...], sw_ref[...], preferred_element_type=jnp.float32)
    ge = geff_ref[...]
    y = y + sb_ref[...] + ge[:, 0:1] * hg1_ref[...] + ge[:, 1:2] * hg2_ref[...]
    y_ref[...] = y


def _stageF(xf, geff, hg_all, sw, sb, T, TB, nblk):
    specs = [
        pl.BlockSpec((TB, MOE_DIM), lambda g: (g, 0)),
        pl.BlockSpec((TB, 2), lambda g: (g, 0)),
        pl.BlockSpec((TB, OUT_CH), lambda g: (g, 0)),
        pl.BlockSpec((TB, OUT_CH), lambda g: (nblk + g, 0)),
        pl.BlockSpec((MOE_DIM, OUT_CH), lambda g: (0, 0)),
        pl.BlockSpec((1, OUT_CH), lambda g: (0, 0)),
    ]
    out_spec = pl.BlockSpec((TB, OUT_CH), lambda g: (g, 0))
    return pl.pallas_call(
        _stageF_body, grid=(nblk,), in_specs=specs, out_specs=out_spec,
        out_shape=jax.ShapeDtypeStruct((T, OUT_CH), jnp.float32),
    )(xf, geff, hg_all, hg_all, sw, sb)


def kernel(x, conv_w, router_w, expert_w, expert_b, shared_w, shared_b):
    B, Cin, H, W = x.shape
    T = B * H * W
    C = int(CAP_F * TOP_K * T / N_EXPERT)
    Cp = ((C + 127) // 128) * 128
    Np = N_EXPERT * Cp
    TB = RB * W
    nbpb = H // RB
    nblk = B * nbpb
    n_assign = TOP_K * T

    cc = np.arange(MOE_DIM) % IN_CH
    jj = np.arange(MOE_DIM) // IN_CH
    perm = 3 * cc + jj
    rw = router_w[perm]
    ew = expert_w[:, perm, :]
    sw = shared_w[perm]
    wt = jnp.transpose(conv_w[:, 0].reshape(IN_CH, 3, 3, 3), (1, 2, 3, 0))
    wt = wt.reshape(27, IN_CH).astype(jnp.bfloat16)

    xT = jnp.transpose(x, (0, 2, 3, 1)).reshape(T, Cin)
    xf, idx, gate, win, hist = _stageA(xT, wt, rw, T, W, nbpb, nblk, TB)
    slots, geff = _stageB(hist, idx, win, gate, C, T, TB, nblk)

    e_all = jnp.transpose(idx).reshape(n_assign)
    s_all = jnp.transpose(slots).reshape(n_assign)
    t_all = jnp.concatenate([jnp.arange(T, dtype=jnp.int32)] * TOP_K)

    tfs = _stageC(e_all, s_all, t_all, C, Cp, Np, n_assign)
    disp = _stageD(tfs[:Np], xf, T, Np)
    h = _stageE(disp, ew, expert_b.reshape(N_EXPERT, 1, OUT_CH), Cp, Np)
    hg_all = _stageG(e_all, s_all, h, C, Cp, Np, n_assign)
    y_flat = _stageF(xf, geff, hg_all, sw, shared_b.reshape(1, OUT_CH),
                     T, TB, nblk)

    return jnp.transpose(y_flat.reshape(B, H, W, OUT_CH), (0, 3, 1, 2))
